# baseline (speedup 1.0000x reference)
import functools

import jax
import jax.numpy as jnp
from jax import lax
from jax.experimental import pallas as pl
from jax.experimental.pallas import tpu as pltpu
from jax.experimental.pallas import tpu_sc as plsc

_H_BLK = 512


def _make_sc_gating(Bt, H, E):
    info = plsc.get_sparse_core_info()
    NC, NS, L = info.num_cores, info.num_subcores, info.num_lanes

    def body(x_hbm, gwt_hbm, s_hbm, xv, gwv, sv):
        wid = lax.axis_index("s") * NC + lax.axis_index("c")

        @pl.when(wid < Bt)
        def _():
            pltpu.sync_copy(x_hbm.at[wid], xv)
            pltpu.sync_copy(gwt_hbm, gwv)
            lanes = lax.iota(jnp.int32, L)

            def allsum(v, buf):
                for sh in (8, 4, 2, 1):
                    buf[...] = v
                    v = v + plsc.load_gather(buf, [(lanes + sh) & (L - 1)])
                return v

            v0 = jnp.full((L,), -3.0e38, jnp.float32)
            v1 = jnp.full((L,), -3.0e38, jnp.float32)
            sel0 = jnp.zeros((L,), jnp.int32)
            for e in range(E):
                def dot_step(i, acc, e=e):
                    return acc + xv[pl.ds(i * L, L)] * gwv[e, pl.ds(i * L, L)]
                acc = lax.fori_loop(0, H // L, dot_step,
                                    jnp.zeros((L,), jnp.float32))
                logit = allsum(acc, sv)
                is_new = logit > v0
                v1 = jnp.where(is_new, v0, jnp.maximum(v1, logit))
                v0 = jnp.where(is_new, logit, v0)
                sel0 = jnp.where(is_new, e, sel0)
            w0 = 1.0 / (1.0 + jnp.exp(v1 - v0))
            sv[...] = jnp.where(sel0 != 0, w0, 0.0)
            pltpu.sync_copy(sv, s_hbm.at[wid])

    return pl.kernel(
        body,
        mesh=plsc.VectorSubcoreMesh(core_axis_name="c", subcore_axis_name="s"),
        out_type=jax.ShapeDtypeStruct((Bt, 16), jnp.float32),
        scratch_types=[
            pltpu.VMEM((H,), jnp.float32),
            pltpu.VMEM((E, H), jnp.float32),
            pltpu.VMEM((16,), jnp.float32),
        ],
        compiler_params=pltpu.CompilerParams(needs_layout_passes=False),
    )


def _mm_step(x_ref, w1_ref, w2_ref, o_ref):
    j = pl.program_id(1)
    xj1 = x_ref[:, pl.ds(2 * j * _H_BLK, _H_BLK)]
    xj2 = x_ref[:, pl.ds((2 * j + 1) * _H_BLK, _H_BLK)]
    part = (jnp.dot(xj1, w1_ref[0], preferred_element_type=jnp.float32)
            + jnp.dot(xj2, w2_ref[0], preferred_element_type=jnp.float32))

    @pl.when(j == 0)
    def _init():
        o_ref[0] = part

    @pl.when(j != 0)
    def _acc():
        o_ref[0] += part


def _expert_matmul(x, expert_w):
    Bt, H = x.shape
    D, _, O = expert_w.shape
    return pl.pallas_call(
        _mm_step,
        grid=(D, H // (2 * _H_BLK)),
        in_specs=[
            pl.BlockSpec((Bt, H), lambda i, j: (0, 0)),
            pl.BlockSpec((1, _H_BLK, O), lambda i, j: (i, 2 * j, 0)),
            pl.BlockSpec((1, _H_BLK, O), lambda i, j: (i, 2 * j + 1, 0)),
        ],
        out_specs=pl.BlockSpec((1, Bt, O), lambda i, j: (i, 0, 0)),
        out_shape=jax.ShapeDtypeStruct((D, Bt, O), jnp.float32),
        compiler_params=pltpu.CompilerParams(
            dimension_semantics=("parallel", "arbitrary")),
    )(x, expert_w, expert_w)


def _scale_step(y_ref, s_ref, o_ref):
    o_ref[0] = y_ref[0] * s_ref[:, 0:1]


def _apply_scale(y, s16):
    D, Bt, O = y.shape
    return pl.pallas_call(
        _scale_step,
        grid=(D,),
        in_specs=[
            pl.BlockSpec((1, Bt, O), lambda i: (i, 0, 0)),
            pl.BlockSpec((Bt, 16), lambda i: (0, 0)),
        ],
        out_specs=pl.BlockSpec((1, Bt, O), lambda i: (i, 0, 0)),
        out_shape=jax.ShapeDtypeStruct((D, Bt, O), jnp.float32),
        compiler_params=pltpu.CompilerParams(
            dimension_semantics=("arbitrary",)),
    )(y, s16)


def kernel(inputs, gate_w, expert_w):
    B, S, H = inputs.shape
    D, _, O = expert_w.shape
    E = gate_w.shape[1]
    x = inputs.reshape(B * S, H)
    gwt = gate_w.T
    s16 = _make_sc_gating(B * S, H, E)(x, gwt)
    y = _expert_matmul(x, expert_w)
    out = _apply_scale(y, s16)
    return out.reshape(D, B, S, 1, O)
